# baseline (device time: 130964 ns/iter reference)
import jax
import jax.numpy as jnp
from jax import lax
from jax.experimental import pallas as pl
from jax.experimental.pallas import tpu as pltpu

N_DEV = 8
M = 1024
N = 1024
K = 4096
KB = 4
CHUNK = M // N_DEV


def kernel(dy, W):
    def body(dy_ref, w_ref, out_ref, acc_ref, stage_ref, rs_recv, ag_recv,
             send_sem, rs_sems, ag_sems):
        me = lax.axis_index("i")
        right = lax.rem(me + 1, N_DEV)
        left = lax.rem(me + N_DEV - 1, N_DEV)

        barrier_sem = pltpu.get_barrier_semaphore()
        for nbr in (left, right):
            pl.semaphore_signal(
                barrier_sem, inc=1,
                device_id=(nbr,), device_id_type=pl.DeviceIdType.MESH,
            )
        pl.semaphore_wait(barrier_sem, 2)

        kc = K // KB
        for kb in range(KB):
            a = dy_ref[:, kb * kc:(kb + 1) * kc].astype(jnp.bfloat16)
            b = w_ref[:, kb * kc:(kb + 1) * kc].astype(jnp.bfloat16)
            p = lax.dot_general(
                a, b, (((1,), (1,)), ((), ())),
                preferred_element_type=jnp.float32,
            )
            if kb == 0:
                acc_ref[...] = p
            else:
                acc_ref[...] += p

        for h in range(N_DEV - 1):
            send_idx = lax.rem(me + N_DEV - h, N_DEV)
            recv_idx = lax.rem(me + N_DEV - h - 1, N_DEV)
            stage_ref[0] = acc_ref[pl.ds(send_idx * CHUNK, CHUNK), :]
            rdma = pltpu.make_async_remote_copy(
                src_ref=stage_ref.at[0],
                dst_ref=rs_recv.at[h],
                send_sem=send_sem.at[0],
                recv_sem=rs_sems.at[h],
                device_id=(right,),
                device_id_type=pl.DeviceIdType.MESH,
            )
            rdma.start()
            rdma.wait()
            acc_ref[pl.ds(recv_idx * CHUNK, CHUNK), :] += rs_recv[h]

        own = lax.rem(me + 1, N_DEV)
        out_ref[pl.ds(own * CHUNK, CHUNK), :] = acc_ref[pl.ds(own * CHUNK, CHUNK), :]

        for h in range(N_DEV - 1):
            send_idx = lax.rem(own + N_DEV - h, N_DEV)
            recv_idx = lax.rem(own + N_DEV - h - 1, N_DEV)
            stage_ref[0] = out_ref[pl.ds(send_idx * CHUNK, CHUNK), :]
            rdma = pltpu.make_async_remote_copy(
                src_ref=stage_ref.at[0],
                dst_ref=ag_recv.at[h],
                send_sem=send_sem.at[0],
                recv_sem=ag_sems.at[h],
                device_id=(right,),
                device_id_type=pl.DeviceIdType.MESH,
            )
            rdma.start()
            rdma.wait()
            out_ref[pl.ds(recv_idx * CHUNK, CHUNK), :] = ag_recv[h]

    return pl.pallas_call(
        body,
        out_shape=jax.ShapeDtypeStruct((M, N), jnp.float32),
        in_specs=[
            pl.BlockSpec(memory_space=pltpu.VMEM),
            pl.BlockSpec(memory_space=pltpu.VMEM),
        ],
        out_specs=pl.BlockSpec(memory_space=pltpu.VMEM),
        scratch_shapes=[
            pltpu.VMEM((M, N), jnp.float32),
            pltpu.VMEM((1, CHUNK, N), jnp.float32),
            pltpu.VMEM((N_DEV - 1, CHUNK, N), jnp.float32),
            pltpu.VMEM((N_DEV - 1, CHUNK, N), jnp.float32),
            pltpu.SemaphoreType.DMA((1,)),
            pltpu.SemaphoreType.DMA((N_DEV - 1,)),
            pltpu.SemaphoreType.DMA((N_DEV - 1,)),
        ],
        compiler_params=pltpu.CompilerParams(collective_id=0),
    )(dy, W)


# device time: 76957 ns/iter; 1.7018x vs baseline; 1.7018x over previous
import jax
import jax.numpy as jnp
from jax import lax
from jax.experimental import pallas as pl
from jax.experimental.pallas import tpu as pltpu

N_DEV = 8
M = 1024
N = 1024
K = 4096
KB = 4


def kernel(dy, W):
    def body(dy_ref, w_ref, out_ref, acc_ref,
             s512, s256, s128, r512, r256, r128, a128, a256, a512,
             send_sem, recv_sems):
        me = lax.axis_index("i")
        z = lax.shift_right_logical(me, 2) & 1
        y = lax.shift_right_logical(me, 1) & 1
        x = (me ^ lax.shift_right_logical(me, 1)) & 1

        p_z = me ^ 4
        p_y = (me & 4) | ((me & 3) ^ 3)
        p_x = (me & 4) | ((me & 3) ^ 1)

        barrier_sem = pltpu.get_barrier_semaphore()
        for nbr in (p_x, p_y, p_z):
            pl.semaphore_signal(
                barrier_sem, inc=1,
                device_id=(nbr,), device_id_type=pl.DeviceIdType.MESH,
            )
        pl.semaphore_wait(barrier_sem, 3)

        kc = K // KB
        for kb in range(KB):
            a = dy_ref[:, kb * kc:(kb + 1) * kc].astype(jnp.bfloat16)
            b = w_ref[:, kb * kc:(kb + 1) * kc].astype(jnp.bfloat16)
            p = lax.dot_general(
                a, b, (((1,), (1,)), ((), ())),
                preferred_element_type=jnp.float32,
            )
            if kb == 0:
                acc_ref[...] = p
            else:
                acc_ref[...] += p

        def exchange(send_buf, recv_buf, stage, partner, src_off, rows):
            send_buf[...] = acc_ref[pl.ds(src_off, rows), :].astype(jnp.bfloat16)
            rdma = pltpu.make_async_remote_copy(
                src_ref=send_buf,
                dst_ref=recv_buf,
                send_sem=send_sem.at[0],
                recv_sem=recv_sems.at[stage],
                device_id=(partner,),
                device_id_type=pl.DeviceIdType.MESH,
            )
            rdma.start()
            rdma.wait()

        my_b1 = z * 512
        exchange(s512, r512, 0, p_z, (1 - z) * 512, 512)
        acc_ref[pl.ds(my_b1, 512), :] += r512[...].astype(jnp.float32)

        my_b2 = my_b1 + y * 256
        exchange(s256, r256, 1, p_y, my_b1 + (1 - y) * 256, 256)
        acc_ref[pl.ds(my_b2, 256), :] += r256[...].astype(jnp.float32)

        my_b3 = my_b2 + x * 128
        exchange(s128, r128, 2, p_x, my_b2 + (1 - x) * 128, 128)
        acc_ref[pl.ds(my_b3, 128), :] += r128[...].astype(jnp.float32)

        out_ref[pl.ds(my_b3, 128), :] = acc_ref[pl.ds(my_b3, 128), :]

        def gather(send_buf, recv_buf, stage, partner, my_off, other_off, rows):
            send_buf[...] = out_ref[pl.ds(my_off, rows), :].astype(jnp.bfloat16)
            rdma = pltpu.make_async_remote_copy(
                src_ref=send_buf,
                dst_ref=recv_buf,
                send_sem=send_sem.at[0],
                recv_sem=recv_sems.at[stage],
                device_id=(partner,),
                device_id_type=pl.DeviceIdType.MESH,
            )
            rdma.start()
            rdma.wait()
            out_ref[pl.ds(other_off, rows), :] = recv_buf[...].astype(jnp.float32)

        gather(s128, a128, 3, p_x, my_b3, my_b2 + (1 - x) * 128, 128)
        gather(s256, a256, 4, p_y, my_b2, my_b1 + (1 - y) * 256, 256)
        gather(s512, a512, 5, p_z, my_b1, (1 - z) * 512, 512)

    return pl.pallas_call(
        body,
        out_shape=jax.ShapeDtypeStruct((M, N), jnp.float32),
        in_specs=[
            pl.BlockSpec(memory_space=pltpu.VMEM),
            pl.BlockSpec(memory_space=pltpu.VMEM),
        ],
        out_specs=pl.BlockSpec(memory_space=pltpu.VMEM),
        scratch_shapes=[
            pltpu.VMEM((M, N), jnp.float32),
            pltpu.VMEM((512, N), jnp.bfloat16),
            pltpu.VMEM((256, N), jnp.bfloat16),
            pltpu.VMEM((128, N), jnp.bfloat16),
            pltpu.VMEM((512, N), jnp.bfloat16),
            pltpu.VMEM((256, N), jnp.bfloat16),
            pltpu.VMEM((128, N), jnp.bfloat16),
            pltpu.VMEM((128, N), jnp.bfloat16),
            pltpu.VMEM((256, N), jnp.bfloat16),
            pltpu.VMEM((512, N), jnp.bfloat16),
            pltpu.SemaphoreType.DMA((1,)),
            pltpu.SemaphoreType.DMA((6,)),
        ],
        compiler_params=pltpu.CompilerParams(collective_id=0),
    )(dy, W)


# device time: 52460 ns/iter; 2.4965x vs baseline; 1.4670x over previous
import jax
import jax.numpy as jnp
from jax import lax
from jax.experimental import pallas as pl
from jax.experimental.pallas import tpu as pltpu

N_DEV = 8
M = 1024
N = 1024
K = 4096
KB = 4

SLAB_COLS = ((0, 384), (384, 768), (768, 1024))
SLAB_AXES = ((0, 1, 2), (1, 2, 0), (2, 0, 1))


def kernel(dy, W):
    def body(dy_ref, w_ref, out_ref, acc_ref, *bufs):
        send_sems = bufs[21]
        recv_sems = bufs[22]

        me = lax.axis_index("i")
        zb = lax.shift_right_logical(me, 2) & 1
        yb = lax.shift_right_logical(me, 1) & 1
        xb = (me ^ lax.shift_right_logical(me, 1)) & 1

        p_z = me ^ 4
        p_y = (me & 4) | ((me & 3) ^ 3)
        p_x = (me & 4) | ((me & 3) ^ 1)
        bit = {0: zb, 1: yb, 2: xb}
        partner = {0: p_z, 1: p_y, 2: p_x}

        kc = K // KB
        for kb in range(KB):
            a = dy_ref[:, kb * kc:(kb + 1) * kc].astype(jnp.bfloat16)
            b = w_ref[:, kb * kc:(kb + 1) * kc].astype(jnp.bfloat16)
            p = lax.dot_general(
                a, b, (((1,), (1,)), ((), ())),
                preferred_element_type=jnp.float32,
            )
            if kb == 0:
                acc_ref[...] = p
            else:
                acc_ref[...] += p

        barrier_sem = pltpu.get_barrier_semaphore()
        for nbr in (p_x, p_y, p_z):
            pl.semaphore_signal(
                barrier_sem, inc=1,
                device_id=(nbr,), device_id_type=pl.DeviceIdType.MESH,
            )
        pl.semaphore_wait(barrier_sem, 3)

        plans = []
        for s in range(3):
            a0, a1, a2 = SLAB_AXES[s]
            b0, b1, b2 = bit[a0], bit[a1], bit[a2]
            p0, p1, p2 = partner[a0], partner[a1], partner[a2]
            my_b1 = b0 * 512
            my_b2 = my_b1 + b1 * 256
            my_b3 = my_b2 + b2 * 128
            plans.append([
                ("rs", 512, p0, (1 - b0) * 512, my_b1),
                ("rs", 256, p1, my_b1 + (1 - b1) * 256, my_b2),
                ("rs", 128, p2, my_b2 + (1 - b2) * 128, my_b3),
                ("ag", 128, p2, my_b3, my_b2 + (1 - b2) * 128),
                ("ag", 256, p1, my_b2, my_b1 + (1 - b1) * 256),
                ("ag", 512, p0, my_b1, (1 - b0) * 512),
            ])

        def slab_bufs(s, t):
            send_ref = bufs[s * 7]
            recv_ref = bufs[s * 7 + 1 + t]
            return send_ref, recv_ref

        def issue(s, t):
            phase, rows, pid, send_off, _ = plans[s][t]
            c0, c1 = SLAB_COLS[s]
            send_ref, recv_ref = slab_bufs(s, t)
            src = acc_ref if phase == "rs" else out_ref
            send_ref[0:rows, :] = src[pl.ds(send_off, rows), c0:c1].astype(
                jnp.bfloat16)
            rdma = pltpu.make_async_remote_copy(
                src_ref=send_ref.at[0:rows],
                dst_ref=recv_ref,
                send_sem=send_sems.at[s],
                recv_sem=recv_sems.at[s * 6 + t],
                device_id=(pid,),
                device_id_type=pl.DeviceIdType.MESH,
            )
            rdma.start()
            return rdma

        def complete(s, t, rdma):
            phase, rows, _, _, recv_off = plans[s][t]
            c0, c1 = SLAB_COLS[s]
            _, recv_ref = slab_bufs(s, t)
            rdma.wait()
            if phase == "rs":
                acc_ref[pl.ds(recv_off, rows), c0:c1] += recv_ref[...].astype(
                    jnp.float32)
            else:
                out_ref[pl.ds(recv_off, rows), c0:c1] = recv_ref[...].astype(
                    jnp.float32)
            if t == 2:
                _, _, _, _, b3 = plans[s][2]
                out_ref[pl.ds(b3, 128), c0:c1] = acc_ref[pl.ds(b3, 128), c0:c1]

        inflight = [issue(s, 0) for s in range(3)]
        for t in range(1, 6):
            for s in range(3):
                complete(s, t - 1, inflight[s])
                inflight[s] = issue(s, t)
        for s in range(3):
            complete(s, 5, inflight[s])

    scratch = [pltpu.VMEM((M, N), jnp.float32)]
    for s in range(3):
        cw = SLAB_COLS[s][1] - SLAB_COLS[s][0]
        scratch.append(pltpu.VMEM((512, cw), jnp.bfloat16))
        for rows in (512, 256, 128, 128, 256, 512):
            scratch.append(pltpu.VMEM((rows, cw), jnp.bfloat16))
    scratch.append(pltpu.SemaphoreType.DMA((3,)))
    scratch.append(pltpu.SemaphoreType.DMA((18,)))

    return pl.pallas_call(
        body,
        out_shape=jax.ShapeDtypeStruct((M, N), jnp.float32),
        in_specs=[
            pl.BlockSpec(memory_space=pltpu.VMEM),
            pl.BlockSpec(memory_space=pltpu.VMEM),
        ],
        out_specs=pl.BlockSpec(memory_space=pltpu.VMEM),
        scratch_shapes=scratch,
        compiler_params=pltpu.CompilerParams(collective_id=0),
    )(dy, W)
